# baseline (speedup 1.0000x reference)
import functools

import jax
import jax.numpy as jnp
import numpy as np
from jax import lax
from jax.experimental import pallas as pl
from jax.experimental.pallas import tpu as pltpu
from jax.experimental.pallas import tpu_sc as plsc

_NUM_SC_CORES = 2
_NUM_SC_SUBCORES = 16
_NUM_WORKERS = _NUM_SC_CORES * _NUM_SC_SUBCORES


def _pe_table(seq_len, d_model):
    pe = np.zeros((seq_len, d_model), dtype=np.float32)
    position = np.arange(0, seq_len).astype(np.float32)[:, None]
    div_term = np.exp(
        np.arange(0, d_model, 2).astype(np.float32) * -(np.log(10000.0) / d_model)
    )
    pe[:, 0::2] = np.sin(position * div_term)
    pe[:, 1::2] = np.cos(position * div_term)
    return pe


def _sc_gather(emb_padded, idx_flat):
    n = idx_flat.shape[0]
    d = emb_padded.shape[1]
    b_per_w = n // _NUM_WORKERS
    mesh = plsc.VectorSubcoreMesh(core_axis_name="c", subcore_axis_name="s")

    chunk = 400
    n_chunks = b_per_w // chunk

    @functools.partial(
        pl.kernel,
        mesh=mesh,
        out_type=jax.ShapeDtypeStruct((n, d), jnp.float32),
        scratch_types=[
            pltpu.VMEM((b_per_w,), jnp.int32),
            pltpu.VMEM((2, chunk, d), jnp.float32),
            pltpu.SemaphoreType.DMA,
            pltpu.SemaphoreType.DMA,
        ],
    )
    def gather_kernel(table_hbm, idx_hbm, out_hbm, idx_v, rows_v, sem0, sem1):
        sems = [sem0, sem1]
        wid = lax.axis_index("s") * _NUM_SC_CORES + lax.axis_index("c")
        base = wid * b_per_w
        pltpu.sync_copy(idx_hbm.at[pl.ds(base, b_per_w)], idx_v)
        copies = [None, None]
        copies[0] = pltpu.async_copy(
            table_hbm.at[idx_v.at[pl.ds(0, chunk)]], rows_v.at[0], sems[0]
        )
        for c in range(n_chunks):
            cur, nxt = c % 2, (c + 1) % 2
            copies[cur].wait()
            if c + 1 < n_chunks:
                copies[nxt] = pltpu.async_copy(
                    table_hbm.at[idx_v.at[pl.ds((c + 1) * chunk, chunk)]],
                    rows_v.at[nxt],
                    sems[nxt],
                )
            pltpu.sync_copy(
                rows_v.at[cur], out_hbm.at[pl.ds(base + c * chunk, chunk)]
            )

    return gather_kernel(emb_padded, idx_flat)


def _tc_body(h_ref, pe_ref, wt_ref, b_ref, o_ref, *, s_per_blk, batch):
    for j in range(s_per_blk):
        z = jnp.tanh(h_ref[pl.ds(j * batch, batch), :64] + pe_ref[j])
        o = lax.dot_general(
            wt_ref[...],
            z,
            (((1,), (1,)), ((), ())),
            preferred_element_type=jnp.float32,
        )
        o_ref[j] = o + b_ref[...]


def _tc_body_aliased(h_ref, pe_ref, wt_ref, b_ref, prev_ref, o_ref, *, s_per_blk, batch):
    del prev_ref
    _tc_body(h_ref, pe_ref, wt_ref, b_ref, o_ref, s_per_blk=s_per_blk, batch=batch)


def kernel(x, emb_table, W, b):
    batch, seq_len = x.shape
    vocab, d_model = emb_table.shape
    n_tok = batch * seq_len

    d_pad = 128
    emb_padded = jnp.pad(emb_table, ((0, 0), (0, d_pad - d_model)))
    idx_flat = x.T.reshape(n_tok).astype(jnp.int32)
    half_s = seq_len // 2
    half_tok = half_s * batch
    hidden_a = _sc_gather(emb_padded, idx_flat[:half_tok])
    hidden_b = _sc_gather(emb_padded, idx_flat[half_tok:])

    pe = jnp.asarray(_pe_table(seq_len, d_model))
    pe3_a = pe[:half_s].reshape(half_s, 1, d_model)
    pe3_b = pe[half_s:].reshape(seq_len - half_s, 1, d_model)
    w_t = W.T
    b_col = b.reshape(vocab, 1)

    s_per_blk = 5
    half_grid = half_s // s_per_blk
    out_shape = jax.ShapeDtypeStruct((seq_len, vocab, batch), jnp.float32)
    body = functools.partial(_tc_body, s_per_blk=s_per_blk, batch=batch)
    body2 = functools.partial(_tc_body_aliased, s_per_blk=s_per_blk, batch=batch)
    common_in_specs = [
        pl.BlockSpec((s_per_blk * batch, d_pad), lambda g: (g, 0)),
        pl.BlockSpec((s_per_blk, 1, d_model), lambda g: (g, 0, 0)),
        pl.BlockSpec((vocab, d_model), lambda g: (0, 0)),
        pl.BlockSpec((vocab, 1), lambda g: (0, 0)),
    ]
    compiler_params = pltpu.CompilerParams(dimension_semantics=("parallel",))

    out_a = pl.pallas_call(
        body,
        grid=(half_grid,),
        in_specs=common_in_specs,
        out_specs=pl.BlockSpec((s_per_blk, vocab, batch), lambda g: (g, 0, 0)),
        out_shape=out_shape,
        compiler_params=compiler_params,
    )(hidden_a, pe3_a, w_t, b_col)

    out_phys = pl.pallas_call(
        body2,
        grid=(half_grid,),
        in_specs=common_in_specs
        + [pl.BlockSpec(memory_space=pltpu.MemorySpace.HBM)],
        out_specs=pl.BlockSpec(
            (s_per_blk, vocab, batch), lambda g, hg=half_grid: (g + hg, 0, 0)
        ),
        out_shape=out_shape,
        input_output_aliases={4: 0},
        compiler_params=compiler_params,
    )(hidden_b, pe3_b, w_t, b_col, out_a)

    return out_phys.transpose(2, 0, 1)
